# baseline (speedup 1.0000x reference)
import jax
import jax.numpy as jnp
from jax import lax
from jax.experimental import pallas as pl
from jax.experimental.pallas import tpu as pltpu
from jax.experimental.pallas import tpu_sc as plsc

_NUM_ROWS = 16384 * 50
_D = 64
_NW = 32
_ROWS_PER_W = _NUM_ROWS // _NW
_CHUNK = 512
_CHUNKS_PER_W = _ROWS_PER_W // _CHUNK
_IB = 128
_NSTREAM = _CHUNK // _IB


def _make_kernel():
    mesh = plsc.VectorSubcoreMesh(core_axis_name="c", subcore_axis_name="s")

    @pl.kernel(
        mesh=mesh,
        out_type=jax.ShapeDtypeStruct((_NUM_ROWS, _D), jnp.float32),
        compiler_params=pltpu.CompilerParams(use_tc_tiling_on_sc=False),
        scratch_types=[
            pltpu.VMEM((_CHUNK,), jnp.int32),
            pltpu.VMEM((_CHUNK,), jnp.int32),
            pltpu.VMEM((_CHUNK, _D), jnp.float32),
            pltpu.VMEM((_CHUNK, _D), jnp.float32),
            pltpu.VMEM((_CHUNKS_PER_W * 16,), jnp.int32),
            pltpu.VMEM((_D,), jnp.float32),
            pltpu.SemaphoreType.DMA,
            pltpu.SemaphoreType.DMA,
            pltpu.SemaphoreType.DMA,
            pltpu.SemaphoreType.DMA,
            pltpu.SemaphoreType.DMA,
            pltpu.SemaphoreType.DMA,
        ],
    )
    def k(table_hbm, idx_hbm, out_hbm,
          idx_v0, idx_v1, rows_v0, rows_v1, accs_v, zrow_v,
          sem_g0, sem_g1, sem_i0, sem_i1, sem_o0, sem_o1):
        wid = lax.axis_index("s") * 2 + lax.axis_index("c")
        base = wid * _ROWS_PER_W

        idx_v = (idx_v0, idx_v1)
        rows_v = (rows_v0, rows_v1)
        sem_g = (sem_g0, sem_g1)
        sem_i = (sem_i0, sem_i1)
        sem_o = (sem_o0, sem_o1)

        zeros16 = jnp.zeros((16,), jnp.float32)
        for cc in range(_D // 16):
            zrow_v[pl.ds(cc * 16, 16)] = zeros16

        def fire_gathers_local(p):
            return [
                pltpu.async_copy(
                    table_hbm.at[idx_v[p].at[pl.ds(j * _IB, _IB)]],
                    rows_v[p].at[pl.ds(j * _IB, _IB)],
                    sem_g[p],
                )
                for j in range(_NSTREAM)
            ]

        def fire_idx(p, g):
            return pltpu.async_copy(
                idx_hbm.at[pl.ds(base + g * _CHUNK, _CHUNK)], idx_v[p],
                sem_i[p])

        def scan_chunk(p, g):
            vregs = [
                idx_v[p][pl.ds(kk * 16, 16)] for kk in range(_CHUNK // 16)
            ]
            accmin = vregs[0]
            for v in vregs[1:]:
                accmin = jnp.minimum(accmin, v)
            accs_v[pl.ds(g * 16, 16)] = accmin

        pltpu.sync_copy(idx_hbm.at[pl.ds(base, _CHUNK)], idx_v0)
        h_g = {0: fire_gathers_local(0)}
        h_i = {1: fire_idx(1, 1)}
        h_o = {}
        for g in range(_CHUNKS_PER_W):
            p = g % 2
            q = 1 - p
            for h in h_g.pop(g):
                h.wait()
            scan_chunk(p, g)
            if g >= 1:
                h_o.pop(g - 1).wait()
            if g + 1 < _CHUNKS_PER_W:
                h_i.pop(g + 1).wait()
                h_g[g + 1] = fire_gathers_local(q)
            if g + 2 < _CHUNKS_PER_W:
                h_i[g + 2] = fire_idx(p, g + 2)
            h_o[g] = pltpu.async_copy(
                rows_v[p], out_hbm.at[pl.ds(base + g * _CHUNK, _CHUNK)],
                sem_o[p])
        h_o.pop(_CHUNKS_PER_W - 1).wait()

        def fix_chunk(g, carry):
            accmin = accs_v[pl.ds(g * 16, 16)]
            has_zero = accmin[0] == 0
            for ll in range(1, 16):
                has_zero = jnp.logical_or(has_zero, accmin[ll] == 0)

            @pl.when(has_zero)
            def _():
                row0 = base + g * _CHUNK
                pltpu.sync_copy(idx_hbm.at[pl.ds(row0, _CHUNK)], idx_v0)

                def fix_group(gg, c2):
                    v = idx_v0[pl.ds(gg * 16, 16)]
                    for ll in range(16):

                        @pl.when(v[ll] == 0)
                        def _():
                            pltpu.sync_copy(
                                zrow_v, out_hbm.at[row0 + gg * 16 + ll])
                    return c2

                lax.fori_loop(0, _CHUNK // 16, fix_group, 0)
            return carry

        lax.fori_loop(0, _CHUNKS_PER_W, fix_chunk, 0)

    return k


_kernel_cache = []


def kernel(x, table):
    if not _kernel_cache:
        _kernel_cache.append(_make_kernel())
    idx = x.reshape(_NUM_ROWS).astype(jnp.int32)
    out = _kernel_cache[0](table, idx)
    return out.reshape(x.shape[0], x.shape[1], _D)


# speedup vs baseline: 1.0041x; 1.0041x over previous
import jax
import jax.numpy as jnp
from jax import lax
from jax.experimental import pallas as pl
from jax.experimental.pallas import tpu as pltpu
from jax.experimental.pallas import tpu_sc as plsc

_NUM_ROWS = 16384 * 50
_D = 64
_NW = 32
_ROWS_PER_W = _NUM_ROWS // _NW
_CHUNK = 640
_CHUNKS_PER_W = _ROWS_PER_W // _CHUNK
_IB = 128
_NSTREAM = _CHUNK // _IB


def _make_kernel():
    mesh = plsc.VectorSubcoreMesh(core_axis_name="c", subcore_axis_name="s")

    @pl.kernel(
        mesh=mesh,
        out_type=jax.ShapeDtypeStruct((_NUM_ROWS, _D), jnp.float32),
        compiler_params=pltpu.CompilerParams(use_tc_tiling_on_sc=False),
        scratch_types=[
            pltpu.VMEM((_CHUNK,), jnp.int32),
            pltpu.VMEM((_CHUNK,), jnp.int32),
            pltpu.VMEM((_CHUNK, _D), jnp.float32),
            pltpu.VMEM((_CHUNK, _D), jnp.float32),
            pltpu.VMEM((_CHUNKS_PER_W * 16,), jnp.int32),
            pltpu.VMEM((_D,), jnp.float32),
            pltpu.SemaphoreType.DMA,
            pltpu.SemaphoreType.DMA,
            pltpu.SemaphoreType.DMA,
            pltpu.SemaphoreType.DMA,
            pltpu.SemaphoreType.DMA,
            pltpu.SemaphoreType.DMA,
        ],
    )
    def k(table_hbm, idx_hbm, out_hbm,
          idx_v0, idx_v1, rows_v0, rows_v1, accs_v, zrow_v,
          sem_g0, sem_g1, sem_i0, sem_i1, sem_o0, sem_o1):
        wid = lax.axis_index("s") * 2 + lax.axis_index("c")
        base = wid * _ROWS_PER_W

        idx_v = (idx_v0, idx_v1)
        rows_v = (rows_v0, rows_v1)
        sem_g = (sem_g0, sem_g1)
        sem_i = (sem_i0, sem_i1)
        sem_o = (sem_o0, sem_o1)

        zeros16 = jnp.zeros((16,), jnp.float32)
        for cc in range(_D // 16):
            zrow_v[pl.ds(cc * 16, 16)] = zeros16

        def fire_gathers_local(p):
            return [
                pltpu.async_copy(
                    table_hbm.at[idx_v[p].at[pl.ds(j * _IB, _IB)]],
                    rows_v[p].at[pl.ds(j * _IB, _IB)],
                    sem_g[p],
                )
                for j in range(_NSTREAM)
            ]

        def fire_idx(p, g):
            return pltpu.async_copy(
                idx_hbm.at[pl.ds(base + g * _CHUNK, _CHUNK)], idx_v[p],
                sem_i[p])

        def scan_chunk(p, g):
            vregs = [
                idx_v[p][pl.ds(kk * 16, 16)] for kk in range(_CHUNK // 16)
            ]
            accmin = vregs[0]
            for v in vregs[1:]:
                accmin = jnp.minimum(accmin, v)
            accs_v[pl.ds(g * 16, 16)] = accmin

        pltpu.sync_copy(idx_hbm.at[pl.ds(base, _CHUNK)], idx_v0)
        h_g = {0: fire_gathers_local(0)}
        h_i = {1: fire_idx(1, 1)}
        h_o = {}
        for g in range(_CHUNKS_PER_W):
            p = g % 2
            q = 1 - p
            for h in h_g.pop(g):
                h.wait()
            scan_chunk(p, g)
            if g >= 1:
                h_o.pop(g - 1).wait()
            if g + 1 < _CHUNKS_PER_W:
                h_i.pop(g + 1).wait()
                h_g[g + 1] = fire_gathers_local(q)
            if g + 2 < _CHUNKS_PER_W:
                h_i[g + 2] = fire_idx(p, g + 2)
            h_o[g] = pltpu.async_copy(
                rows_v[p], out_hbm.at[pl.ds(base + g * _CHUNK, _CHUNK)],
                sem_o[p])
        h_o.pop(_CHUNKS_PER_W - 1).wait()

        def fix_chunk(g, carry):
            accmin = accs_v[pl.ds(g * 16, 16)]
            has_zero = accmin[0] == 0
            for ll in range(1, 16):
                has_zero = jnp.logical_or(has_zero, accmin[ll] == 0)

            @pl.when(has_zero)
            def _():
                row0 = base + g * _CHUNK
                pltpu.sync_copy(idx_hbm.at[pl.ds(row0, _CHUNK)], idx_v0)

                def fix_group(gg, c2):
                    v = idx_v0[pl.ds(gg * 16, 16)]
                    for ll in range(16):

                        @pl.when(v[ll] == 0)
                        def _():
                            pltpu.sync_copy(
                                zrow_v, out_hbm.at[row0 + gg * 16 + ll])
                    return c2

                lax.fori_loop(0, _CHUNK // 16, fix_group, 0)
            return carry

        lax.fori_loop(0, _CHUNKS_PER_W, fix_chunk, 0)

    return k


_kernel_cache = []


def kernel(x, table):
    if not _kernel_cache:
        _kernel_cache.append(_make_kernel())
    idx = x.reshape(_NUM_ROWS).astype(jnp.int32)
    out = _kernel_cache[0](table, idx)
    return out.reshape(x.shape[0], x.shape[1], _D)
